# baseline (speedup 1.0000x reference)
import functools

import jax
import jax.numpy as jnp
import numpy as np
from jax.experimental import pallas as pl
from jax.experimental.pallas import tpu as pltpu

_B = 1024
_NU = 10000
_NB = 10000
_NC = 500
_NG = 1000
_NF = 4
_ED = 32
_CD = _NF * _ED
_NITER = 2
_KB = 200
_NK = _NU // _KB


def _facet_mats():
    d = jax.lax.broadcasted_iota(jnp.int32, (_NF, _CD), 1)
    f = jax.lax.broadcasted_iota(jnp.int32, (_NF, _CD), 0)
    M4 = jnp.where(d // _ED == f, 1.0, 0.0).astype(jnp.float32)
    d2 = jax.lax.broadcasted_iota(jnp.int32, (_CD, _CD), 0)
    e2 = jax.lax.broadcasted_iota(jnp.int32, (_CD, _CD), 1)
    P = jnp.where(d2 // _ED == e2 // _ED, 1.0, 0.0).astype(jnp.float32)
    return M4, P


def _bf(x):
    return x.astype(jnp.bfloat16)


def _body(u_ref, b_ref, unu_ref, unb_ref, bnu_ref, bnb_ref,
          unc_ref, ung_ref, bnc_ref, bng_ref,
          Wu_ref, bu_ref, Wb_ref, bb_ref, Wc_ref, bc_ref, Wg_ref, bg_ref,
          out_ref,
          ue_a, be_a, unu_a, unb_a, unc_a, ung_a, bnu_a, bnb_a, bnc_a,
          bng_a):
    k = pl.program_id(0)

    Wu = _bf(Wu_ref[...])
    Wb = _bf(Wb_ref[...])

    def chunk(W, x_ref):
        return jax.lax.dot_general(
            W, _bf(x_ref[...]),
            dimension_numbers=(((0,), (0,)), ((), ())),
            preferred_element_type=jnp.float32)

    def bias_col(b_ref_):
        return jnp.broadcast_to(b_ref_[...], (_CD, _B))

    @pl.when(k == 0)
    def _():
        ue_a[...] = bias_col(bu_ref)
        be_a[...] = bias_col(bb_ref)
        unu_a[...] = bias_col(bu_ref)
        unb_a[...] = bias_col(bb_ref)
        bnu_a[...] = bias_col(bu_ref)
        bnb_a[...] = bias_col(bb_ref)
        Wc = _bf(Wc_ref[...])
        Wg = _bf(Wg_ref[...])
        unc_a[...] = chunk(Wc, unc_ref) + bias_col(bc_ref)
        bnc_a[...] = chunk(Wc, bnc_ref) + bias_col(bc_ref)
        ung_a[...] = chunk(Wg, ung_ref) + bias_col(bg_ref)
        bng_a[...] = chunk(Wg, bng_ref) + bias_col(bg_ref)

    ue_a[...] += chunk(Wu, u_ref)
    be_a[...] += chunk(Wb, b_ref)
    unu_a[...] += chunk(Wu, unu_ref)
    unb_a[...] += chunk(Wb, unb_ref)
    bnu_a[...] += chunk(Wu, bnu_ref)
    bnb_a[...] += chunk(Wb, bnb_ref)

    @pl.when(k == _NK - 1)
    def _():
        M4, P = _facet_mats()

        def fsum(x):
            return jnp.dot(M4, x, preferred_element_type=jnp.float32)

        def fsb(x):
            return jnp.dot(P, x, preferred_element_type=jnp.float32)

        def homo(t, n):
            ab = jax.nn.sigmoid(fsb(t * n) * (1.0 / np.sqrt(_ED)))
            return ab * t + (1.0 - ab) * n

        def hete(t, zs):
            u = t
            for _ in range(_NITER):
                s = [fsb(u * z) for z in zs]
                m = jnp.maximum(jnp.maximum(s[0], s[1]),
                                jnp.maximum(s[2], s[3]))
                es = [jnp.exp(si - m) for si in s]
                den = es[0] + es[1] + es[2] + es[3]
                agg = (es[0] / den) * zs[0]
                for ei, z in zip(es[1:], zs[1:]):
                    agg = agg + (ei / den) * z
                u = t + agg
                inv = 1.0 / (jnp.sqrt(fsb(u * u)) + 1e-9)
                u = u * inv
            return u

        ue = ue_a[...]
        be = be_a[...]
        u_homo = [homo(ue, unu_a[...]), homo(ue, unb_a[...]),
                  homo(ue, unc_a[...]), homo(ue, ung_a[...])]
        b_homo = [homo(be, bnu_a[...]), homo(be, bnb_a[...]),
                  homo(be, bnc_a[...]), homo(be, bng_a[...])]
        uu = hete(ue, u_homo)
        ub = hete(ue, b_homo)
        logit = fsum(uu * ub)
        m = jnp.max(logit, axis=0, keepdims=True)
        e = jnp.exp(logit - m)
        out_ref[...] = e / jnp.sum(e, axis=0, keepdims=True)


@functools.partial(jax.jit, static_argnames=())
def kernel(users, businesses, un_user, un_biz, un_city, un_cat,
           bn_user, bn_biz, bn_city, bn_cat,
           W_user, b_user, W_biz, b_biz, W_city, b_city, W_cat, b_cat):
    grid = (_NK,)

    def kspec():
        return pl.BlockSpec((_KB, _B), lambda k: (k, 0))

    def wspec():
        return pl.BlockSpec((_KB, _CD), lambda k: (k, 0))

    def const_spec(shape):
        return pl.BlockSpec(shape, lambda k: (0, 0))

    in_specs = [
        kspec(), kspec(), kspec(), kspec(), kspec(), kspec(),
        const_spec((_NC, _B)), const_spec((_NG, _B)),
        const_spec((_NC, _B)), const_spec((_NG, _B)),
        wspec(), const_spec((_CD, 1)),
        wspec(), const_spec((_CD, 1)),
        const_spec((_NC, _CD)), const_spec((_CD, 1)),
        const_spec((_NG, _CD)), const_spec((_CD, 1)),
    ]
    out_spec = pl.BlockSpec((_NF, _B), lambda k: (0, 0))
    scratch = [pltpu.VMEM((_CD, _B), jnp.float32) for _ in range(10)]

    out_t = pl.pallas_call(
        _body,
        grid=grid,
        in_specs=in_specs,
        out_specs=out_spec,
        out_shape=jax.ShapeDtypeStruct((_NF, _B), jnp.float32),
        scratch_shapes=scratch,
        compiler_params=pltpu.CompilerParams(
            dimension_semantics=("arbitrary",)),
    )(users.T, businesses.T, un_user.T, un_biz.T, bn_user.T, bn_biz.T,
      un_city.T, un_cat.T, bn_city.T, bn_cat.T,
      W_user, b_user.reshape(_CD, 1), W_biz, b_biz.reshape(_CD, 1),
      W_city, b_city.reshape(_CD, 1), W_cat, b_cat.reshape(_CD, 1))
    return out_t.T


# speedup vs baseline: 1.0757x; 1.0757x over previous
import functools

import jax
import jax.numpy as jnp
import numpy as np
from jax.experimental import pallas as pl
from jax.experimental.pallas import tpu as pltpu

_B = 1024
_NU = 10000
_NB = 10000
_NC = 500
_NG = 1000
_NF = 4
_ED = 32
_CD = _NF * _ED
_NITER = 2
_KB = 624
_NK = -(-_NU // _KB)
_REM = _NU - (_NK - 1) * _KB


def _facet_mats():
    d = jax.lax.broadcasted_iota(jnp.int32, (_NF, _CD), 1)
    f = jax.lax.broadcasted_iota(jnp.int32, (_NF, _CD), 0)
    M4 = jnp.where(d // _ED == f, 1.0, 0.0).astype(jnp.float32)
    d2 = jax.lax.broadcasted_iota(jnp.int32, (_CD, _CD), 0)
    e2 = jax.lax.broadcasted_iota(jnp.int32, (_CD, _CD), 1)
    P = jnp.where(d2 // _ED == e2 // _ED, 1.0, 0.0).astype(jnp.float32)
    return M4, P


def _bf(x):
    return x.astype(jnp.bfloat16)


def _body(u_ref, b_ref, unu_ref, unb_ref, bnu_ref, bnb_ref,
          unc_ref, ung_ref, bnc_ref, bng_ref,
          Wu_ref, bu_ref, Wb_ref, bb_ref, Wc_ref, bc_ref, Wg_ref, bg_ref,
          out_ref,
          ue_a, be_a, unu_a, unb_a, unc_a, ung_a, bnu_a, bnb_a, bnc_a,
          bng_a):
    k = pl.program_id(0)

    Wu = _bf(Wu_ref[...])
    Wb = _bf(Wb_ref[...])

    def chunk(W, x_ref):
        return jax.lax.dot_general(
            W, _bf(x_ref[...]),
            dimension_numbers=(((0,), (0,)), ((), ())),
            preferred_element_type=jnp.float32)

    def bias_col(b_ref_):
        return jnp.broadcast_to(b_ref_[...], (_CD, _B))

    @pl.when(k == 0)
    def _():
        ue_a[...] = bias_col(bu_ref)
        be_a[...] = bias_col(bb_ref)
        unu_a[...] = bias_col(bu_ref)
        unb_a[...] = bias_col(bb_ref)
        bnu_a[...] = bias_col(bu_ref)
        bnb_a[...] = bias_col(bb_ref)
        Wc = _bf(Wc_ref[...])
        Wg = _bf(Wg_ref[...])
        unc_a[...] = chunk(Wc, unc_ref) + bias_col(bc_ref)
        bnc_a[...] = chunk(Wc, bnc_ref) + bias_col(bc_ref)
        ung_a[...] = chunk(Wg, ung_ref) + bias_col(bg_ref)
        bng_a[...] = chunk(Wg, bng_ref) + bias_col(bg_ref)

    @pl.when(k < _NK - 1)
    def _():
        ue_a[...] += chunk(Wu, u_ref)
        be_a[...] += chunk(Wb, b_ref)
        unu_a[...] += chunk(Wu, unu_ref)
        unb_a[...] += chunk(Wb, unb_ref)
        bnu_a[...] += chunk(Wu, bnu_ref)
        bnb_a[...] += chunk(Wb, bnb_ref)

    @pl.when(k == _NK - 1)
    def _():
        rows_x = jax.lax.broadcasted_iota(jnp.int32, (_KB, _B), 0)
        rows_w = jax.lax.broadcasted_iota(jnp.int32, (_KB, _CD), 0)

        def mchunk(W, x_ref):
            Wm = jnp.where(rows_w < _REM, W, jnp.zeros_like(W))
            xm = jnp.where(rows_x < _REM, _bf(x_ref[...]),
                           jnp.zeros((_KB, _B), jnp.bfloat16))
            return jax.lax.dot_general(
                Wm, xm, dimension_numbers=(((0,), (0,)), ((), ())),
                preferred_element_type=jnp.float32)

        ue_a[...] += mchunk(Wu, u_ref)
        be_a[...] += mchunk(Wb, b_ref)
        unu_a[...] += mchunk(Wu, unu_ref)
        unb_a[...] += mchunk(Wb, unb_ref)
        bnu_a[...] += mchunk(Wu, bnu_ref)
        bnb_a[...] += mchunk(Wb, bnb_ref)

    @pl.when(k == _NK - 1)
    def _():
        M4, P = _facet_mats()

        def fsum(x):
            return jnp.dot(M4, x, preferred_element_type=jnp.float32)

        def fsb(x):
            return jnp.dot(P, x, preferred_element_type=jnp.float32)

        def homo(t, n):
            ab = jax.nn.sigmoid(fsb(t * n) * (1.0 / np.sqrt(_ED)))
            return ab * t + (1.0 - ab) * n

        def hete(t, zs):
            u = t
            for _ in range(_NITER):
                s = [fsb(u * z) for z in zs]
                m = jnp.maximum(jnp.maximum(s[0], s[1]),
                                jnp.maximum(s[2], s[3]))
                es = [jnp.exp(si - m) for si in s]
                den = es[0] + es[1] + es[2] + es[3]
                agg = (es[0] / den) * zs[0]
                for ei, z in zip(es[1:], zs[1:]):
                    agg = agg + (ei / den) * z
                u = t + agg
                inv = 1.0 / (jnp.sqrt(fsb(u * u)) + 1e-9)
                u = u * inv
            return u

        ue = ue_a[...]
        be = be_a[...]
        u_homo = [homo(ue, unu_a[...]), homo(ue, unb_a[...]),
                  homo(ue, unc_a[...]), homo(ue, ung_a[...])]
        b_homo = [homo(be, bnu_a[...]), homo(be, bnb_a[...]),
                  homo(be, bnc_a[...]), homo(be, bng_a[...])]
        uu = hete(ue, u_homo)
        ub = hete(ue, b_homo)
        logit = fsum(uu * ub)
        m = jnp.max(logit, axis=0, keepdims=True)
        e = jnp.exp(logit - m)
        out_ref[...] = e / jnp.sum(e, axis=0, keepdims=True)


@functools.partial(jax.jit, static_argnames=())
def kernel(users, businesses, un_user, un_biz, un_city, un_cat,
           bn_user, bn_biz, bn_city, bn_cat,
           W_user, b_user, W_biz, b_biz, W_city, b_city, W_cat, b_cat):
    grid = (_NK,)

    def kspec():
        return pl.BlockSpec((_KB, _B), lambda k: (k, 0))

    def wspec():
        return pl.BlockSpec((_KB, _CD), lambda k: (k, 0))

    def const_spec(shape):
        return pl.BlockSpec(shape, lambda k: (0, 0))

    in_specs = [
        kspec(), kspec(), kspec(), kspec(), kspec(), kspec(),
        const_spec((_NC, _B)), const_spec((_NG, _B)),
        const_spec((_NC, _B)), const_spec((_NG, _B)),
        wspec(), const_spec((_CD, 1)),
        wspec(), const_spec((_CD, 1)),
        const_spec((_NC, _CD)), const_spec((_CD, 1)),
        const_spec((_NG, _CD)), const_spec((_CD, 1)),
    ]
    out_spec = pl.BlockSpec((_NF, _B), lambda k: (0, 0))
    scratch = [pltpu.VMEM((_CD, _B), jnp.float32) for _ in range(10)]

    out_t = pl.pallas_call(
        _body,
        grid=grid,
        in_specs=in_specs,
        out_specs=out_spec,
        out_shape=jax.ShapeDtypeStruct((_NF, _B), jnp.float32),
        scratch_shapes=scratch,
        compiler_params=pltpu.CompilerParams(
            dimension_semantics=("arbitrary",)),
    )(users.T, businesses.T, un_user.T, un_biz.T, bn_user.T, bn_biz.T,
      un_city.T, un_cat.T, bn_city.T, bn_cat.T,
      W_user, b_user.reshape(_CD, 1), W_biz, b_biz.reshape(_CD, 1),
      W_city, b_city.reshape(_CD, 1), W_cat, b_cat.reshape(_CD, 1))
    return out_t.T


# speedup vs baseline: 1.0822x; 1.0061x over previous
import functools

import jax
import jax.numpy as jnp
import numpy as np
from jax.experimental import pallas as pl
from jax.experimental.pallas import tpu as pltpu

_B = 1024
_NU = 10000
_NB = 10000
_NC = 500
_NG = 1000
_NF = 4
_ED = 32
_CD = _NF * _ED
_NITER = 2
_KB = 400
_NK = _NU // _KB


def _facet_mats():
    d = jax.lax.broadcasted_iota(jnp.int32, (_NF, _CD), 1)
    f = jax.lax.broadcasted_iota(jnp.int32, (_NF, _CD), 0)
    M4 = jnp.where(d // _ED == f, 1.0, 0.0).astype(jnp.float32)
    d2 = jax.lax.broadcasted_iota(jnp.int32, (_CD, _NF), 0)
    f2 = jax.lax.broadcasted_iota(jnp.int32, (_CD, _NF), 1)
    M4T = jnp.where(d2 // _ED == f2, 1.0, 0.0).astype(jnp.float32)
    return M4, M4T


def _bf(x):
    return x.astype(jnp.bfloat16)


def _body(u_ref, b_ref, unu_ref, unb_ref, bnu_ref, bnb_ref,
          unc_ref, ung_ref, bnc_ref, bng_ref,
          Wu_ref, bu_ref, Wb_ref, bb_ref, Wc_ref, bc_ref, Wg_ref, bg_ref,
          out_ref,
          ue_a, be_a, unu_a, unb_a, unc_a, ung_a, bnu_a, bnb_a, bnc_a,
          bng_a):
    k = pl.program_id(0)

    Wu = _bf(Wu_ref[...])
    Wb = _bf(Wb_ref[...])

    def chunk(W, x_ref):
        return jax.lax.dot_general(
            W, _bf(x_ref[...]),
            dimension_numbers=(((0,), (0,)), ((), ())),
            preferred_element_type=jnp.float32)

    def bias_col(b_ref_):
        return jnp.broadcast_to(b_ref_[...], (_CD, _B))

    @pl.when(k == 0)
    def _():
        ue_a[...] = bias_col(bu_ref)
        be_a[...] = bias_col(bb_ref)
        unu_a[...] = bias_col(bu_ref)
        unb_a[...] = bias_col(bb_ref)
        bnu_a[...] = bias_col(bu_ref)
        bnb_a[...] = bias_col(bb_ref)
        Wc = _bf(Wc_ref[...])
        Wg = _bf(Wg_ref[...])
        unc_a[...] = chunk(Wc, unc_ref) + bias_col(bc_ref)
        bnc_a[...] = chunk(Wc, bnc_ref) + bias_col(bc_ref)
        ung_a[...] = chunk(Wg, ung_ref) + bias_col(bg_ref)
        bng_a[...] = chunk(Wg, bng_ref) + bias_col(bg_ref)

    ue_a[...] += chunk(Wu, u_ref)
    be_a[...] += chunk(Wb, b_ref)
    unu_a[...] += chunk(Wu, unu_ref)
    unb_a[...] += chunk(Wb, unb_ref)
    bnu_a[...] += chunk(Wu, bnu_ref)
    bnb_a[...] += chunk(Wb, bnb_ref)

    @pl.when(k == _NK - 1)
    def _():
        M4, M4T = _facet_mats()

        def fsum(x):
            return jnp.dot(M4, x, preferred_element_type=jnp.float32)

        def fbcast(s):
            return jnp.dot(M4T, s, preferred_element_type=jnp.float32)

        def homo(t, n):
            a = jax.nn.sigmoid(fsum(t * n) * (1.0 / np.sqrt(_ED)))
            ab = fbcast(a)
            return ab * t + (1.0 - ab) * n

        def hete(t, zs):
            u = t
            for _ in range(_NITER):
                s = [fsum(u * z) for z in zs]
                m = jnp.maximum(jnp.maximum(s[0], s[1]),
                                jnp.maximum(s[2], s[3]))
                es = [jnp.exp(si - m) for si in s]
                den = es[0] + es[1] + es[2] + es[3]
                agg = fbcast(es[0] / den) * zs[0]
                for ei, z in zip(es[1:], zs[1:]):
                    agg = agg + fbcast(ei / den) * z
                u = t + agg
                inv = 1.0 / (jnp.sqrt(fsum(u * u)) + 1e-9)
                u = u * fbcast(inv)
            return u

        ue = ue_a[...]
        be = be_a[...]
        u_homo = [homo(ue, unu_a[...]), homo(ue, unb_a[...]),
                  homo(ue, unc_a[...]), homo(ue, ung_a[...])]
        b_homo = [homo(be, bnu_a[...]), homo(be, bnb_a[...]),
                  homo(be, bnc_a[...]), homo(be, bng_a[...])]
        uu = hete(ue, u_homo)
        ub = hete(ue, b_homo)
        logit = fsum(uu * ub)
        m = jnp.max(logit, axis=0, keepdims=True)
        e = jnp.exp(logit - m)
        out_ref[...] = e / jnp.sum(e, axis=0, keepdims=True)


@functools.partial(jax.jit, static_argnames=())
def kernel(users, businesses, un_user, un_biz, un_city, un_cat,
           bn_user, bn_biz, bn_city, bn_cat,
           W_user, b_user, W_biz, b_biz, W_city, b_city, W_cat, b_cat):
    grid = (_NK,)

    def kspec():
        return pl.BlockSpec((_KB, _B), lambda k: (k, 0))

    def wspec():
        return pl.BlockSpec((_KB, _CD), lambda k: (k, 0))

    def const_spec(shape):
        return pl.BlockSpec(shape, lambda k: (0, 0))

    in_specs = [
        kspec(), kspec(), kspec(), kspec(), kspec(), kspec(),
        const_spec((_NC, _B)), const_spec((_NG, _B)),
        const_spec((_NC, _B)), const_spec((_NG, _B)),
        wspec(), const_spec((_CD, 1)),
        wspec(), const_spec((_CD, 1)),
        const_spec((_NC, _CD)), const_spec((_CD, 1)),
        const_spec((_NG, _CD)), const_spec((_CD, 1)),
    ]
    out_spec = pl.BlockSpec((_NF, _B), lambda k: (0, 0))
    scratch = [pltpu.VMEM((_CD, _B), jnp.float32) for _ in range(10)]

    out_t = pl.pallas_call(
        _body,
        grid=grid,
        in_specs=in_specs,
        out_specs=out_spec,
        out_shape=jax.ShapeDtypeStruct((_NF, _B), jnp.float32),
        scratch_shapes=scratch,
        compiler_params=pltpu.CompilerParams(
            dimension_semantics=("arbitrary",)),
    )(users.T, businesses.T, un_user.T, un_biz.T, bn_user.T, bn_biz.T,
      un_city.T, un_cat.T, bn_city.T, bn_cat.T,
      W_user, b_user.reshape(_CD, 1), W_biz, b_biz.reshape(_CD, 1),
      W_city, b_city.reshape(_CD, 1), W_cat, b_cat.reshape(_CD, 1))
    return out_t.T
